# Initial kernel scaffold; baseline (speedup 1.0000x reference)
import jax
import jax.numpy as jnp
from jax.experimental import pallas as pl


def kernel(x, edge_index, batch, Wl1, Wr1, b1, Ws1, bs1, Wl2, Wr2, b2, Ws2, bs2, Wl3, Wr3, b3, Ws3, bs3, Wm1, bm1, Wm2, bm2):
    raise NotImplementedError("write your pallas kernel here")


# speedup vs baseline: 6.7783x; 6.7783x over previous
import functools

import jax
import jax.numpy as jnp
from jax import lax
from jax.experimental import pallas as pl
from jax.experimental.pallas import tpu as pltpu
from jax.experimental.pallas import tpu_sc as plsc

_NC = 2
_NS = 16
_NW = _NC * _NS


def _mm_body(x_ref, w_ref, b_ref, o_ref):
    o_ref[...] = (
        jnp.dot(x_ref[...], w_ref[...], preferred_element_type=jnp.float32,
                precision=lax.Precision.HIGHEST)
        + b_ref[...]
    )


def _mm(x, w, b, bm=1000):
    m, k = x.shape
    n = w.shape[1]
    return pl.pallas_call(
        _mm_body,
        grid=(m // bm,),
        in_specs=[
            pl.BlockSpec((bm, k), lambda i: (i, 0)),
            pl.BlockSpec((k, n), lambda i: (0, 0)),
            pl.BlockSpec((1, n), lambda i: (0, 0)),
        ],
        out_specs=pl.BlockSpec((bm, n), lambda i: (i, 0)),
        out_shape=jax.ShapeDtypeStruct((m, n), jnp.float32),
    )(x, w, b.reshape(1, -1))


def _layer_body(z_ref, p_ref, c_ref, w_ref, b_ref, o_ref):
    cnt = c_ref[0, :, 0:1] + c_ref[1, :, 0:1]
    inv = 1.0 / jnp.maximum(cnt, 1.0)
    h = jnp.maximum(z_ref[...] + (p_ref[0] + p_ref[1]) * inv, 0.0)
    o_ref[...] = (
        jnp.dot(h, w_ref[...], preferred_element_type=jnp.float32,
                precision=lax.Precision.HIGHEST) + b_ref[...]
    )


def _layer(z, p, c, w, b, bm=1000):
    m, d = z.shape
    n = w.shape[1]
    return pl.pallas_call(
        _layer_body,
        grid=(m // bm,),
        in_specs=[
            pl.BlockSpec((bm, d), lambda i: (i, 0)),
            pl.BlockSpec((2, bm, d), lambda i: (0, i, 0)),
            pl.BlockSpec((2, bm, 16), lambda i: (0, i, 0)),
            pl.BlockSpec((d, n), lambda i: (0, 0)),
            pl.BlockSpec((1, n), lambda i: (0, 0)),
        ],
        out_specs=pl.BlockSpec((bm, n), lambda i: (i, 0)),
        out_shape=jax.ShapeDtypeStruct((m, n), jnp.float32),
    )(z, p, c, w, b.reshape(1, -1))


def _pool_body(z_ref, p_ref, c_ref, batch_ref, o_ref, *, num_graphs):
    cnt = c_ref[0, :, 0:1] + c_ref[1, :, 0:1]
    inv = 1.0 / jnp.maximum(cnt, 1.0)
    h = jnp.maximum(z_ref[...] + (p_ref[0] + p_ref[1]) * inv, 0.0)

    i = pl.program_id(0)

    @pl.when(i == 0)
    def _():
        o_ref[...] = jnp.full_like(o_ref[...], -jnp.inf)

    b_ids = batch_ref[...]
    g_iota = lax.broadcasted_iota(jnp.int32, (num_graphs, 1), 0)
    for g in range(num_graphs):
        v = jnp.max(jnp.where(b_ids == g, h, -jnp.inf), axis=0, keepdims=True)
        upd = jnp.where(g_iota == g, v, -jnp.inf)
        o_ref[...] = jnp.maximum(o_ref[...], upd)


def _pool(z, p, c, batch2d, num_graphs, bm=1000):
    m, d = z.shape
    return pl.pallas_call(
        functools.partial(_pool_body, num_graphs=num_graphs),
        grid=(m // bm,),
        in_specs=[
            pl.BlockSpec((bm, d), lambda i: (i, 0)),
            pl.BlockSpec((2, bm, d), lambda i: (0, i, 0)),
            pl.BlockSpec((2, bm, 16), lambda i: (0, i, 0)),
            pl.BlockSpec((bm, 1), lambda i: (i, 0)),
        ],
        out_specs=pl.BlockSpec((num_graphs, d), lambda i: (0, 0)),
        out_shape=jax.ShapeDtypeStruct((num_graphs, d), jnp.float32),
    )(z, p, c, batch2d)


def _mlp_body(p_ref, w1_ref, b1_ref, w2_ref, b2_ref, o_ref):
    hh = jnp.maximum(
        jnp.dot(p_ref[...], w1_ref[...], preferred_element_type=jnp.float32,
                precision=lax.Precision.HIGHEST)
        + b1_ref[...],
        0.0,
    )
    o_ref[...] = (
        jnp.dot(hh, w2_ref[...], preferred_element_type=jnp.float32,
                precision=lax.Precision.HIGHEST)
        + b2_ref[...]
    )


def _mlp(pooled, w1, b1, w2, b2):
    g, d = pooled.shape
    nout = w2.shape[1]
    return pl.pallas_call(
        _mlp_body,
        out_shape=jax.ShapeDtypeStruct((g, nout), jnp.float32),
    )(pooled, w1, b1.reshape(1, -1), w2, b2.reshape(1, -1))


def _edge_cnt(dst, n):
    e = dst.shape[0]
    ept = e // _NW
    ch = 80
    nchunk = ept // ch
    n_pad = ((n + _NS * ch - 1) // (_NS * ch)) * (_NS * ch)
    nrow = n_pad // 128
    assert nrow % 8 == 0
    nwr = nrow // 8

    mesh = plsc.VectorSubcoreMesh(
        core_axis_name="c", subcore_axis_name="s", num_cores=_NC)

    def body(dst_hbm, zhist_hbm, zhist1_hbm, cnt_hbm, dst_v, iota_v,
             hist1_v, hist_v, cbuf_v, cnt_sp):
        cid = lax.axis_index("c")
        sid = lax.axis_index("s")

        pltpu.sync_copy(zhist1_hbm, hist1_v)
        for k in range(nrow // 16):
            iota_v[pl.ds(16 * k, 16)] = lax.iota(jnp.int32, 16) + 16 * k

        @pl.when(sid < nwr)
        def _():
            pltpu.sync_copy(zhist_hbm.at[pl.ds(0, 8)], cbuf_v)
            pltpu.sync_copy(cbuf_v, cnt_sp.at[pl.ds(sid * 8, 8)])
        plsc.subcore_barrier()

        tile = cid * _NS + sid
        ones16 = jnp.ones((16,), jnp.float32)

        @pl.loop(0, nchunk)
        def _(j):
            base = tile * ept + j * ch
            pltpu.sync_copy(dst_hbm.at[pl.ds(base, ch)], dst_v)
            for k in range(ch // 16):
                idx = dst_v[pl.ds(16 * k, 16)]
                plsc.addupdate_scatter(hist1_v, [idx], ones16)

        @pl.loop(0, nrow)
        def _(r):
            for k in range(8):
                hist_v[r, pl.ds(16 * k, 16)] = hist1_v[pl.ds(128 * r + 16 * k,
                                                             16)]
        pltpu.sync_copy(hist_v, cnt_sp.at[iota_v], add=True)
        plsc.subcore_barrier()

        @pl.when(sid < nwr)
        def _():
            pltpu.sync_copy(cnt_sp.at[pl.ds(sid * 8, 8)], cbuf_v)
            pltpu.sync_copy(cbuf_v, cnt_hbm.at[cid, pl.ds(sid * 8, 8)])

    run = pl.kernel(
        body,
        out_type=(jax.ShapeDtypeStruct((_NC, nrow, 128), jnp.float32),),
        mesh=mesh,
        compiler_params=pltpu.CompilerParams(needs_layout_passes=False),
        scratch_types=(pltpu.VMEM((ch,), jnp.int32),
                       pltpu.VMEM((nrow,), jnp.int32),
                       pltpu.VMEM((nrow * 128,), jnp.float32),
                       pltpu.VMEM((nrow, 128), jnp.float32),
                       pltpu.VMEM((8, 128), jnp.float32),
                       pltpu.VMEM_SHARED((nrow, 128), jnp.float32)))
    zhist = jnp.zeros((nrow, 128), jnp.float32)
    zhist1 = jnp.zeros((nrow * 128,), jnp.float32)
    c = run(dst, zhist, zhist1)[0]
    return jnp.broadcast_to(
        c.reshape(_NC, -1)[:, :n, None], (_NC, n, 16))


def _edge_agg_sc(y, src, dst):
    n, d = y.shape
    e = src.shape[0]
    ept = e // _NW
    ch = 80
    nchunk = ept // ch
    n_pad = ((n + _NS * ch - 1) // (_NS * ch)) * (_NS * ch)
    rps = n_pad // _NS

    out_type = [jax.ShapeDtypeStruct((_NC, n_pad, d), jnp.float32)]

    assert rps % ch == 0
    nio = rps // ch

    scratch = [
        pltpu.VMEM((ch,), jnp.int32),
        pltpu.VMEM((ch,), jnp.int32),
        pltpu.VMEM((ch, d), jnp.float32),
        pltpu.VMEM((ch,), jnp.int32),
        pltpu.VMEM((ch,), jnp.int32),
        pltpu.VMEM((ch, d), jnp.float32),
        pltpu.SemaphoreType.DMA,
        pltpu.SemaphoreType.DMA,
        pltpu.VMEM_SHARED((n_pad, d), jnp.float32),
    ]

    mesh = plsc.VectorSubcoreMesh(
        core_axis_name="c", subcore_axis_name="s", num_cores=_NC)

    assert nchunk % 2 == 1 and nchunk >= 3

    def body(y_hbm, src_hbm, dst_hbm, zrow_hbm, agg_hbm,
             src0_v, dst0_v, rows0_v, src1_v, dst1_v,
             rows1_v, sem0, sem1, agg_sp):
        cid = lax.axis_index("c")
        sid = lax.axis_index("s")
        base_r = sid * rps
        srcb = (src0_v, src1_v)
        dstb = (dst0_v, dst1_v)
        rowsb = (rows0_v, rows1_v)
        semb = (sem0, sem1)

        pltpu.sync_copy(zrow_hbm, rows0_v)
        for k in range(nio):
            pltpu.sync_copy(rows0_v, agg_sp.at[pl.ds(base_r + k * ch, ch)])
        plsc.subcore_barrier()

        tile = cid * _NS + sid

        def start_gather(j, b):
            base = tile * ept + j * ch
            pltpu.sync_copy(src_hbm.at[pl.ds(base, ch)], srcb[b])
            pltpu.sync_copy(dst_hbm.at[pl.ds(base, ch)], dstb[b])
            cp = pltpu.make_async_copy(y_hbm.at[srcb[b]], rowsb[b], semb[b])
            cp.start()

        def finish_scatter(b):
            pltpu.make_async_copy(
                y_hbm.at[srcb[b]], rowsb[b], semb[b]).wait()
            pltpu.sync_copy(rowsb[b], agg_sp.at[dstb[b]], add=True)

        start_gather(0, 0)

        @pl.loop(0, (nchunk - 1) // 2)
        def _(i):
            start_gather(2 * i + 1, 1)
            finish_scatter(0)
            start_gather(2 * i + 2, 0)
            finish_scatter(1)

        finish_scatter(0)
        plsc.subcore_barrier()

        for k in range(nio):
            pltpu.sync_copy(agg_sp.at[pl.ds(base_r + k * ch, ch)], rows0_v)
            pltpu.sync_copy(rows0_v,
                            agg_hbm.at[cid, pl.ds(base_r + k * ch, ch)])

    run = pl.kernel(body, out_type=tuple(out_type), mesh=mesh,
                    scratch_types=tuple(scratch))
    zrow = jnp.zeros((ch, d), jnp.float32)
    return run(y, src, dst, zrow)[0]


def kernel(x, edge_index, batch,
           Wl1, Wr1, b1, Ws1, bs1,
           Wl2, Wr2, b2, Ws2, bs2,
           Wl3, Wr3, b3, Ws3, bs3,
           Wm1, bm1, Wm2, bm2):
    src = edge_index[0]
    dst = edge_index[1]
    h = Wl1.shape[1]

    w1 = jnp.concatenate([Wl1, Wr1 + Ws1], axis=1)
    bias1 = jnp.concatenate([jnp.zeros((h,), jnp.float32), b1 + bs1])
    w2 = jnp.concatenate([Wl2, Wr2 + Ws2], axis=1)
    bias2 = jnp.concatenate([jnp.zeros((h,), jnp.float32), b2 + bs2])
    w3 = jnp.concatenate([Wl3, Wr3 + Ws3], axis=1)
    bias3 = jnp.concatenate([jnp.zeros((h,), jnp.float32), b3 + bs3])

    yz1 = _mm(x, w1, bias1)
    y1 = yz1[:, :h]
    z1 = yz1[:, h:]

    cnt = _edge_cnt(dst, x.shape[0])
    p1 = _edge_agg_sc(y1, src, dst)
    yz2 = _layer(z1, p1, cnt, w2, bias2)
    y2 = yz2[:, :h]
    z2 = yz2[:, h:]

    p2 = _edge_agg_sc(y2, src, dst)
    yz3 = _layer(z2, p2, cnt, w3, bias3)
    y3 = yz3[:, :h]
    z3 = yz3[:, h:]

    p3 = _edge_agg_sc(y3, src, dst)

    batch2d = batch.reshape(-1, 1).astype(jnp.int32)
    pooled = _pool(z3, p3, cnt, batch2d, 64)
    return _mlp(pooled, Wm1, bm1, Wm2, bm2)
